# baseline (speedup 1.0000x reference)
import functools

import jax
import jax.numpy as jnp
from jax.experimental import pallas as pl
from jax.experimental.pallas import tpu as pltpu

_HS = 768
_E = 64
_TOPK = 8
_BT = 4096
_HH = _HS // 2


def _router_body(xa_ref, xb_ref, w_ref, wout_ref, iout_ref):
    w = w_ref[...]
    logits = jax.lax.dot_general(
        w[:, :_HH], xa_ref[...], (((1,), (1,)), ((), ())),
        preferred_element_type=jnp.float32)
    logits = logits + jax.lax.dot_general(
        w[:, _HH:], xb_ref[...], (((1,), (1,)), ((), ())),
        preferred_element_type=jnp.float32)
    m = jnp.max(logits, axis=0, keepdims=True)
    p = jnp.exp(logits - m)
    rdenom = 1.0 / jnp.sum(p, axis=0, keepdims=True)

    eidx = jax.lax.broadcasted_iota(jnp.int32, (_E, _BT), 0)
    vals = p
    for k in range(_TOPK):
        mk = jnp.max(vals, axis=0, keepdims=True)
        hit = vals == mk
        idx = jnp.min(jnp.where(hit, eidx, _E), axis=0, keepdims=True)
        wout_ref[pl.ds(k, 1), :] = mk * rdenom
        iout_ref[pl.ds(k, 1), :] = idx
        vals = jnp.where(eidx == idx, -1.0, vals)


@jax.jit
def _router(xf, W):
    n = xf.shape[0]
    grid = (n // _BT,)
    return pl.pallas_call(
        _router_body,
        grid=grid,
        in_specs=[
            pl.BlockSpec((_BT, _HH), lambda i: (i, 0)),
            pl.BlockSpec((_BT, _HH), lambda i: (i, 1)),
            pl.BlockSpec((_E, _HS), lambda i: (0, 0)),
        ],
        out_specs=[
            pl.BlockSpec((_TOPK, _BT), lambda i: (0, i)),
            pl.BlockSpec((_TOPK, _BT), lambda i: (0, i)),
        ],
        out_shape=[
            jax.ShapeDtypeStruct((_TOPK, n), jnp.float32),
            jax.ShapeDtypeStruct((_TOPK, n), jnp.int32),
        ],
    )(xf, xf, W)


def kernel(x, W):
    xf = x.reshape(-1, x.shape[-1])
    wT, iT = _router(xf, W)
    return (wT.T, iT.T)
